# Initial kernel scaffold; baseline (speedup 1.0000x reference)
import jax
import jax.numpy as jnp
from jax.experimental import pallas as pl


def kernel(queries, memory_keys, memory_identifiers, memory_entity_ids, text_identifiers):
    raise NotImplementedError("write your pallas kernel here")


# speedup vs baseline: 3.9439x; 3.9439x over previous
import functools

import jax
import jax.numpy as jnp
from jax import lax
from jax.experimental import pallas as pl
from jax.experimental.pallas import tpu as pltpu
from jax.experimental.pallas import tpu_sc as plsc

Q = 256
R = 4096
V = 32
D = 128
K_TOP = 32
LARGE_NUMBER = 1e10
NEG = -1e30

RB = 256

NC = 2
NS = 16
LANES = 16
NW = NC * NS
QPW = Q // NW
IPW = QPW * K_TOP
NVEC = IPW // LANES


def _scores_body(q_ref, k_ref, mx_ref, am_ref):
    q = q_ref[...]
    m = None
    a = None
    for v in range(V):
        kv = k_ref[:, v, :]
        s = lax.dot_general(q, kv, (((1,), (1,)), ((), ())),
                            preferred_element_type=jnp.float32)
        if v == 0:
            m = s
            a = jnp.zeros(s.shape, jnp.int32)
        else:
            gt = s > m
            m = jnp.where(gt, s, m)
            a = jnp.where(gt, v, a)
    mx_ref[...] = m
    am_ref[...] = a


def _rowmax(queries, memory_keys):
    return pl.pallas_call(
        _scores_body,
        grid=(R // RB,),
        in_specs=[
            pl.BlockSpec((Q, D), lambda i: (0, 0)),
            pl.BlockSpec((RB, V, D), lambda i: (i, 0, 0)),
        ],
        out_specs=[
            pl.BlockSpec((Q, RB), lambda i: (0, i)),
            pl.BlockSpec((Q, RB), lambda i: (0, i)),
        ],
        out_shape=[
            jax.ShapeDtypeStruct((Q, R), jnp.float32),
            jax.ShapeDtypeStruct((Q, R), jnp.int32),
        ],
        compiler_params=pltpu.CompilerParams(
            dimension_semantics=("arbitrary",)),
    )(queries, memory_keys)


def _topk_body(mx_ref, s_ref, i_ref):
    vals = mx_ref[...]
    col = lax.broadcasted_iota(jnp.int32, (Q, R), 1)
    big = jnp.int32(1 << 30)
    outs_s = []
    outs_i = []
    for _ in range(K_TOP):
        m = jnp.max(vals, axis=1, keepdims=True)
        idx = jnp.min(jnp.where(vals == m, col, big), axis=1, keepdims=True)
        outs_s.append(m)
        outs_i.append(idx)
        vals = jnp.where(col == idx, NEG, vals)
    s_ref[...] = jnp.concatenate(outs_s, axis=1)
    i_ref[...] = jnp.concatenate(outs_i, axis=1)


def _topk(row_max):
    return pl.pallas_call(
        _topk_body,
        out_shape=[
            jax.ShapeDtypeStruct((Q, K_TOP), jnp.float32),
            jax.ShapeDtypeStruct((Q, K_TOP), jnp.int32),
        ],
    )(row_max)


def _sc_retrieve(trs, tri, arg2d, ids2d, ents2d, flat_keys, text_ids):
    mesh = plsc.VectorSubcoreMesh(core_axis_name="c", subcore_axis_name="s",
                                  num_cores=NC, num_subcores=NS)

    @functools.partial(
        pl.kernel,
        out_type=[
            jax.ShapeDtypeStruct((Q * K_TOP,), jnp.float32),
            jax.ShapeDtypeStruct((Q * K_TOP,), jnp.float32),
            jax.ShapeDtypeStruct((Q * K_TOP, D), jnp.float32),
            jax.ShapeDtypeStruct((Q, D), jnp.float32),
            jax.ShapeDtypeStruct((Q * K_TOP,), jnp.int32),
            jax.ShapeDtypeStruct((Q * K_TOP,), jnp.int32),
            jax.ShapeDtypeStruct((Q * K_TOP,), jnp.int32),
        ],
        mesh=mesh,
        compiler_params=pltpu.CompilerParams(needs_layout_passes=False,
                                             use_tc_tiling_on_sc=False),
        scratch_types=[
            pltpu.VMEM((IPW,), jnp.float32),
            pltpu.VMEM((IPW,), jnp.int32),
            pltpu.VMEM((IPW,), jnp.int32),
            pltpu.VMEM((IPW, 16), jnp.int32),
            pltpu.VMEM((IPW, 16), jnp.int32),
            pltpu.VMEM((IPW, 16), jnp.int32),
            pltpu.VMEM((IPW,), jnp.int32),
            pltpu.VMEM((IPW,), jnp.float32),
            pltpu.VMEM((IPW,), jnp.float32),
            pltpu.VMEM((IPW,), jnp.int32),
            pltpu.VMEM((IPW,), jnp.int32),
            pltpu.VMEM((IPW, D), jnp.float32),
            pltpu.VMEM((QPW, D), jnp.float32),
            pltpu.VMEM((QPW,), jnp.int32),
            pltpu.SemaphoreType.DMA,
        ],
    )
    def body(trs_h, tri_h, arg2d_h, ids2d_h, ents2d_h, keys_h, tid_h,
             ms_h, w_h, tv_h, ret_h, ent_h, gti_h, mask_h,
             trs_v, tri_v, rowidx_v, argrows_v, idrows_v, entrows_v,
             gti_v, ms_v, w_v, ent_v, mask_v, keys_v, ret_v, tid_v, sem):
        wid = lax.axis_index("s") * NC + lax.axis_index("c")
        ibase = wid * IPW
        qbase = wid * QPW

        pltpu.sync_copy(trs_h.at[pl.ds(ibase, IPW)], trs_v)
        pltpu.sync_copy(tri_h.at[pl.ds(ibase, IPW)], tri_v)
        pltpu.sync_copy(tid_h.at[pl.ds(qbase, QPW)], tid_v)

        lane = lax.iota(jnp.int32, 16)

        def l1(j, c):
            rid = tri_v[pl.ds(j * LANES, LANES)]
            qg = qbase + j // 2
            rowidx_v[pl.ds(j * LANES, LANES)] = (
                qg * (R // 16) + lax.shift_right_logical(rid, 2 + 2))
            return c
        lax.fori_loop(0, NVEC, l1, 0)
        pltpu.async_copy(arg2d_h.at[rowidx_v], argrows_v, sem).wait()

        def l2(j, c):
            rid = tri_v[pl.ds(j * LANES, LANES)]
            pos = j * LANES + lane
            a = plsc.load_gather(argrows_v, [pos, jnp.bitwise_and(rid, 15)])
            g = rid * V + a
            gti_v[pl.ds(j * LANES, LANES)] = g
            rowidx_v[pl.ds(j * LANES, LANES)] = lax.shift_right_logical(g, 4)
            return c
        lax.fori_loop(0, NVEC, l2, 0)
        pltpu.async_copy(ids2d_h.at[rowidx_v], idrows_v, sem).wait()
        pltpu.async_copy(ents2d_h.at[rowidx_v], entrows_v, sem).wait()
        pltpu.async_copy(keys_h.at[gti_v], keys_v, sem).wait()

        def l3(j, c):
            g = gti_v[pl.ds(j * LANES, LANES)]
            pos = j * LANES + lane
            lane16 = jnp.bitwise_and(g, 15)
            mid = plsc.load_gather(idrows_v, [pos, lane16])
            ent = plsc.load_gather(entrows_v, [pos, lane16])
            ent_v[pl.ds(j * LANES, LANES)] = ent
            tloc = jnp.zeros((16,), jnp.int32) + j // 2
            tb = plsc.load_gather(tid_v, [tloc])
            mk = (mid == tb).astype(jnp.int32)
            mask_v[pl.ds(j * LANES, LANES)] = mk
            s = trs_v[pl.ds(j * LANES, LANES)]
            ms_v[pl.ds(j * LANES, LANES)] = (
                s - mk.astype(jnp.float32) * LARGE_NUMBER)
            return c
        lax.fori_loop(0, NVEC, l3, 0)

        def l4(q, c):
            a = ms_v[pl.ds(q * K_TOP, 16)]
            b = ms_v[pl.ds(q * K_TOP + 16, 16)]
            m = jnp.max(jnp.maximum(a, b))
            ea = jnp.exp(a - m)
            eb = jnp.exp(b - m)
            s = jnp.sum(ea) + jnp.sum(eb)
            w_v[pl.ds(q * K_TOP, 16)] = ea / s
            w_v[pl.ds(q * K_TOP + 16, 16)] = eb / s
            return c
        lax.fori_loop(0, QPW, l4, 0)

        def l5(q, c):
            def inner(k2, acc):
                wk = plsc.load_gather(
                    w_v, [jnp.zeros((16,), jnp.int32) + (q * K_TOP + k2)])
                row = q * K_TOP + k2
                return tuple(
                    acc[ch] + keys_v[row, pl.ds(ch * 16, 16)] * wk
                    for ch in range(D // 16))
            acc0 = tuple(jnp.zeros((16,), jnp.float32)
                         for _ in range(D // 16))
            acc = lax.fori_loop(0, K_TOP, inner, acc0)
            for ch in range(D // 16):
                ret_v[q, pl.ds(ch * 16, 16)] = acc[ch]
            return c
        lax.fori_loop(0, QPW, l5, 0)

        pltpu.sync_copy(ms_v, ms_h.at[pl.ds(ibase, IPW)])
        pltpu.sync_copy(w_v, w_h.at[pl.ds(ibase, IPW)])
        pltpu.sync_copy(keys_v, tv_h.at[pl.ds(ibase, IPW)])
        pltpu.sync_copy(ret_v, ret_h.at[pl.ds(qbase, QPW)])
        pltpu.sync_copy(ent_v, ent_h.at[pl.ds(ibase, IPW)])
        pltpu.sync_copy(gti_v, gti_h.at[pl.ds(ibase, IPW)])
        pltpu.sync_copy(mask_v, mask_h.at[pl.ds(ibase, IPW)])

    return body(trs, tri, arg2d, ids2d, ents2d, flat_keys, text_ids)


def kernel(queries, memory_keys, memory_identifiers, memory_entity_ids,
           text_identifiers):
    flat_keys = memory_keys.reshape(R * V, D)
    ids2d = memory_identifiers.reshape(-1, 16)
    ents2d = memory_entity_ids.reshape(-1, 16)

    row_max, row_arg = _rowmax(queries, memory_keys)
    trs, tri = _topk(row_max)
    arg2d = row_arg.reshape(-1, 16)

    ms, w, tv, ret, ent, gti, mk = _sc_retrieve(
        trs.reshape(-1), tri.reshape(-1), arg2d, ids2d, ents2d, flat_keys,
        text_identifiers)

    ms = ms.reshape(Q, K_TOP)
    w = w.reshape(Q, K_TOP)
    tv = tv.reshape(Q, K_TOP, D)
    ent = ent.reshape(Q, K_TOP)
    gti = gti.reshape(Q, K_TOP)
    mask = mk.reshape(Q, K_TOP).astype(jnp.bool_)
    return (ret, ms, w, tv, ent, gti, mask)


# speedup vs baseline: 4.5768x; 1.1605x over previous
import functools

import jax
import jax.numpy as jnp
from jax import lax
from jax.experimental import pallas as pl
from jax.experimental.pallas import tpu as pltpu
from jax.experimental.pallas import tpu_sc as plsc

Q = 256
R = 4096
V = 32
D = 128
K_TOP = 32
LARGE_NUMBER = 1e10
NEG = -1e30

RB = 256

NC = 2
NS = 16
LANES = 16
NW = NC * NS
QPW = Q // NW
IPW = QPW * K_TOP
NVEC = IPW // LANES


def _scores_body(*refs):
    q_ref = refs[0]
    key_refs = refs[1:1 + V]
    mx_ref, am_ref = refs[1 + V], refs[2 + V]
    q = q_ref[...]
    m = None
    a = None
    for v in range(V):
        kv = key_refs[v][:, 0, 0, :]
        s = lax.dot_general(q, kv, (((1,), (1,)), ((), ())),
                            preferred_element_type=jnp.float32)
        if v == 0:
            m = s
            a = jnp.zeros(s.shape, jnp.int32)
        else:
            gt = s > m
            m = jnp.where(gt, s, m)
            a = jnp.where(gt, v, a)
    mx_ref[...] = m
    am_ref[...] = a


def _keyspec(v):
    return pl.BlockSpec((RB, 1, 1, D), lambda i, _v=v: (i, _v, 0, 0))


def _rowmax(queries, memory_keys):
    return pl.pallas_call(
        _scores_body,
        grid=(R // RB,),
        in_specs=[pl.BlockSpec((Q, D), lambda i: (0, 0))]
        + [_keyspec(v) for v in range(V)],
        out_specs=[
            pl.BlockSpec((Q, RB), lambda i: (0, i)),
            pl.BlockSpec((Q, RB), lambda i: (0, i)),
        ],
        out_shape=[
            jax.ShapeDtypeStruct((Q, R), jnp.float32),
            jax.ShapeDtypeStruct((Q, R), jnp.int32),
        ],
        compiler_params=pltpu.CompilerParams(
            dimension_semantics=("arbitrary",)),
    )(queries, *([memory_keys.reshape(R, V, 1, D)] * V))


def _topk_body(mx_ref, s_ref, i_ref):
    vals = mx_ref[...]
    colf = lax.broadcasted_iota(jnp.int32, (Q, R), 1).astype(jnp.float32)
    bigf = jnp.float32(1e9)
    outs_s = []
    outs_i = []
    for _ in range(K_TOP):
        m = jnp.max(vals, axis=1, keepdims=True)
        eq = vals == m
        idxf = jnp.min(jnp.where(eq, colf, bigf), axis=1, keepdims=True)
        outs_s.append(m)
        outs_i.append(idxf)
        vals = jnp.where(eq, NEG, vals)
    s_ref[...] = jnp.concatenate(outs_s, axis=1)
    i_ref[...] = jnp.concatenate(outs_i, axis=1).astype(jnp.int32)


def _topk(row_max):
    return pl.pallas_call(
        _topk_body,
        out_shape=[
            jax.ShapeDtypeStruct((Q, K_TOP), jnp.float32),
            jax.ShapeDtypeStruct((Q, K_TOP), jnp.int32),
        ],
    )(row_max)


def _sc_retrieve(trs, tri, arg2d, ids2d, ents2d, flat_keys, text_ids):
    mesh = plsc.VectorSubcoreMesh(core_axis_name="c", subcore_axis_name="s",
                                  num_cores=NC, num_subcores=NS)

    @functools.partial(
        pl.kernel,
        out_type=[
            jax.ShapeDtypeStruct((Q * K_TOP,), jnp.float32),
            jax.ShapeDtypeStruct((Q * K_TOP,), jnp.float32),
            jax.ShapeDtypeStruct((Q * K_TOP, D), jnp.float32),
            jax.ShapeDtypeStruct((Q, D), jnp.float32),
            jax.ShapeDtypeStruct((Q * K_TOP,), jnp.int32),
            jax.ShapeDtypeStruct((Q * K_TOP,), jnp.int32),
            jax.ShapeDtypeStruct((Q * K_TOP,), jnp.int32),
        ],
        mesh=mesh,
        compiler_params=pltpu.CompilerParams(needs_layout_passes=False,
                                             use_tc_tiling_on_sc=False),
        scratch_types=[
            pltpu.VMEM((IPW,), jnp.float32),
            pltpu.VMEM((IPW,), jnp.int32),
            pltpu.VMEM((IPW,), jnp.int32),
            pltpu.VMEM((IPW, 16), jnp.int32),
            pltpu.VMEM((IPW, 16), jnp.int32),
            pltpu.VMEM((IPW, 16), jnp.int32),
            pltpu.VMEM((IPW,), jnp.int32),
            pltpu.VMEM((IPW,), jnp.float32),
            pltpu.VMEM((IPW,), jnp.float32),
            pltpu.VMEM((IPW,), jnp.int32),
            pltpu.VMEM((IPW,), jnp.int32),
            pltpu.VMEM((IPW, D), jnp.float32),
            pltpu.VMEM((QPW, D), jnp.float32),
            pltpu.VMEM((QPW,), jnp.int32),
            pltpu.SemaphoreType.DMA,
        ],
    )
    def body(trs_h, tri_h, arg2d_h, ids2d_h, ents2d_h, keys_h, tid_h,
             ms_h, w_h, tv_h, ret_h, ent_h, gti_h, mask_h,
             trs_v, tri_v, rowidx_v, argrows_v, idrows_v, entrows_v,
             gti_v, ms_v, w_v, ent_v, mask_v, keys_v, ret_v, tid_v, sem):
        wid = lax.axis_index("s") * NC + lax.axis_index("c")
        ibase = wid * IPW
        qbase = wid * QPW

        pltpu.sync_copy(trs_h.at[pl.ds(ibase, IPW)], trs_v)
        pltpu.sync_copy(tri_h.at[pl.ds(ibase, IPW)], tri_v)
        pltpu.sync_copy(tid_h.at[pl.ds(qbase, QPW)], tid_v)

        lane = lax.iota(jnp.int32, 16)

        def l1(j, c):
            rid = tri_v[pl.ds(j * LANES, LANES)]
            qg = qbase + j // 2
            rowidx_v[pl.ds(j * LANES, LANES)] = (
                qg * (R // 16) + lax.shift_right_logical(rid, 2 + 2))
            return c
        lax.fori_loop(0, NVEC, l1, 0)
        pltpu.async_copy(arg2d_h.at[rowidx_v], argrows_v, sem).wait()

        def l2(j, c):
            rid = tri_v[pl.ds(j * LANES, LANES)]
            pos = j * LANES + lane
            a = plsc.load_gather(argrows_v, [pos, jnp.bitwise_and(rid, 15)])
            g = rid * V + a
            gti_v[pl.ds(j * LANES, LANES)] = g
            rowidx_v[pl.ds(j * LANES, LANES)] = lax.shift_right_logical(g, 4)
            return c
        lax.fori_loop(0, NVEC, l2, 0)
        pltpu.async_copy(ids2d_h.at[rowidx_v], idrows_v, sem).wait()
        pltpu.async_copy(ents2d_h.at[rowidx_v], entrows_v, sem).wait()
        pltpu.async_copy(keys_h.at[gti_v], keys_v, sem).wait()

        def l3(j, c):
            g = gti_v[pl.ds(j * LANES, LANES)]
            pos = j * LANES + lane
            lane16 = jnp.bitwise_and(g, 15)
            mid = plsc.load_gather(idrows_v, [pos, lane16])
            ent = plsc.load_gather(entrows_v, [pos, lane16])
            ent_v[pl.ds(j * LANES, LANES)] = ent
            tloc = jnp.zeros((16,), jnp.int32) + j // 2
            tb = plsc.load_gather(tid_v, [tloc])
            mk = (mid == tb).astype(jnp.int32)
            mask_v[pl.ds(j * LANES, LANES)] = mk
            s = trs_v[pl.ds(j * LANES, LANES)]
            ms_v[pl.ds(j * LANES, LANES)] = (
                s - mk.astype(jnp.float32) * LARGE_NUMBER)
            return c
        lax.fori_loop(0, NVEC, l3, 0)

        def l4(q, c):
            a = ms_v[pl.ds(q * K_TOP, 16)]
            b = ms_v[pl.ds(q * K_TOP + 16, 16)]
            m = jnp.max(jnp.maximum(a, b))
            ea = jnp.exp(a - m)
            eb = jnp.exp(b - m)
            s = jnp.sum(ea) + jnp.sum(eb)
            w_v[pl.ds(q * K_TOP, 16)] = ea / s
            w_v[pl.ds(q * K_TOP + 16, 16)] = eb / s
            return c
        lax.fori_loop(0, QPW, l4, 0)

        def l5(q, c):
            def inner(k2, acc):
                wk = plsc.load_gather(
                    w_v, [jnp.zeros((16,), jnp.int32) + (q * K_TOP + k2)])
                row = q * K_TOP + k2
                return tuple(
                    acc[ch] + keys_v[row, pl.ds(ch * 16, 16)] * wk
                    for ch in range(D // 16))
            acc0 = tuple(jnp.zeros((16,), jnp.float32)
                         for _ in range(D // 16))
            acc = lax.fori_loop(0, K_TOP, inner, acc0)
            for ch in range(D // 16):
                ret_v[q, pl.ds(ch * 16, 16)] = acc[ch]
            return c
        lax.fori_loop(0, QPW, l5, 0)

        pltpu.sync_copy(ms_v, ms_h.at[pl.ds(ibase, IPW)])
        pltpu.sync_copy(w_v, w_h.at[pl.ds(ibase, IPW)])
        pltpu.sync_copy(keys_v, tv_h.at[pl.ds(ibase, IPW)])
        pltpu.sync_copy(ret_v, ret_h.at[pl.ds(qbase, QPW)])
        pltpu.sync_copy(ent_v, ent_h.at[pl.ds(ibase, IPW)])
        pltpu.sync_copy(gti_v, gti_h.at[pl.ds(ibase, IPW)])
        pltpu.sync_copy(mask_v, mask_h.at[pl.ds(ibase, IPW)])

    return body(trs, tri, arg2d, ids2d, ents2d, flat_keys, text_ids)


def kernel(queries, memory_keys, memory_identifiers, memory_entity_ids,
           text_identifiers):
    flat_keys = memory_keys.reshape(R * V, D)
    ids2d = memory_identifiers.reshape(-1, 16)
    ents2d = memory_entity_ids.reshape(-1, 16)

    row_max, row_arg = _rowmax(queries, memory_keys)
    trs, tri = _topk(row_max)
    arg2d = row_arg.reshape(-1, 16)

    ms, w, tv, ret, ent, gti, mk = _sc_retrieve(
        trs.reshape(-1), tri.reshape(-1), arg2d, ids2d, ents2d, flat_keys,
        text_identifiers)

    ms = ms.reshape(Q, K_TOP)
    w = w.reshape(Q, K_TOP)
    tv = tv.reshape(Q, K_TOP, D)
    ent = ent.reshape(Q, K_TOP)
    gti = gti.reshape(Q, K_TOP)
    mask = mk.reshape(Q, K_TOP).astype(jnp.bool_)
    return (ret, ms, w, tv, ent, gti, mask)


# speedup vs baseline: 4.5987x; 1.0048x over previous
import functools

import jax
import jax.numpy as jnp
from jax import lax
from jax.experimental import pallas as pl
from jax.experimental.pallas import tpu as pltpu
from jax.experimental.pallas import tpu_sc as plsc

Q = 256
R = 4096
V = 32
D = 128
K_TOP = 32
LARGE_NUMBER = 1e10
NEG = -1e30

RB = 256

NC = 2
NS = 16
LANES = 16
NW = NC * NS
QPW = Q // NW
IPW = QPW * K_TOP
NVEC = IPW // LANES


NB = R // RB


def _fused_body(*refs):
    q_ref = refs[0]
    key_refs = refs[1:1 + V]
    am_ref, s_ref, i_ref = refs[1 + V], refs[2 + V], refs[3 + V]
    vals3_ref = refs[4 + V]
    i_blk = pl.program_id(0)

    q = q_ref[...]
    m = None
    a = None
    for v in range(V):
        kv = key_refs[v][:, 0, 0, :]
        s = lax.dot_general(q, kv, (((1,), (1,)), ((), ())),
                            preferred_element_type=jnp.float32)
        if v == 0:
            m = s
            a = jnp.zeros(s.shape, jnp.int32)
        else:
            gt = s > m
            m = jnp.where(gt, s, m)
            a = jnp.where(gt, v, a)
    am_ref[...] = a
    vals3_ref[i_blk] = m

    @pl.when(i_blk == NB - 1)
    def _epilogue():
        vals = vals3_ref[...]
        colf3 = (lax.broadcasted_iota(jnp.int32, (NB, Q, RB), 0) * RB
                 + lax.broadcasted_iota(jnp.int32, (NB, Q, RB), 2)
                 ).astype(jnp.float32)
        bigf = jnp.float32(1e9)
        outs_s = []
        outs_i = []
        for _ in range(K_TOP):
            m1 = jnp.max(vals, axis=0)
            mm = jnp.max(m1, axis=1, keepdims=True)
            eq = vals == mm[None, :, :]
            t = jnp.where(eq, colf3, bigf)
            i1 = jnp.min(t, axis=0)
            idxf = jnp.min(i1, axis=1, keepdims=True)
            outs_s.append(mm)
            outs_i.append(idxf)
            vals = jnp.where(eq, NEG, vals)
        s_ref[...] = jnp.concatenate(outs_s, axis=1)
        i_ref[...] = jnp.concatenate(outs_i, axis=1).astype(jnp.int32)


def _keyspec(v):
    return pl.BlockSpec((RB, 1, 1, D), lambda i, _v=v: (i, _v, 0, 0))


def _scores_topk(queries, memory_keys):
    return pl.pallas_call(
        _fused_body,
        grid=(NB,),
        in_specs=[pl.BlockSpec((Q, D), lambda i: (0, 0))]
        + [_keyspec(v) for v in range(V)],
        out_specs=[
            pl.BlockSpec((Q, RB), lambda i: (0, i)),
            pl.BlockSpec((Q, K_TOP), lambda i: (0, 0)),
            pl.BlockSpec((Q, K_TOP), lambda i: (0, 0)),
        ],
        out_shape=[
            jax.ShapeDtypeStruct((Q, R), jnp.int32),
            jax.ShapeDtypeStruct((Q, K_TOP), jnp.float32),
            jax.ShapeDtypeStruct((Q, K_TOP), jnp.int32),
        ],
        scratch_shapes=[pltpu.VMEM((NB, Q, RB), jnp.float32)],
        compiler_params=pltpu.CompilerParams(
            dimension_semantics=("arbitrary",)),
    )(queries, *([memory_keys.reshape(R, V, 1, D)] * V))


def _sc_retrieve(trs, tri, arg2d, ids2d, ents2d, flat_keys, text_ids):
    mesh = plsc.VectorSubcoreMesh(core_axis_name="c", subcore_axis_name="s",
                                  num_cores=NC, num_subcores=NS)

    @functools.partial(
        pl.kernel,
        out_type=[
            jax.ShapeDtypeStruct((Q * K_TOP,), jnp.float32),
            jax.ShapeDtypeStruct((Q * K_TOP,), jnp.float32),
            jax.ShapeDtypeStruct((Q * K_TOP, D), jnp.float32),
            jax.ShapeDtypeStruct((Q, D), jnp.float32),
            jax.ShapeDtypeStruct((Q * K_TOP,), jnp.int32),
            jax.ShapeDtypeStruct((Q * K_TOP,), jnp.int32),
            jax.ShapeDtypeStruct((Q * K_TOP,), jnp.int32),
        ],
        mesh=mesh,
        compiler_params=pltpu.CompilerParams(needs_layout_passes=False,
                                             use_tc_tiling_on_sc=False),
        scratch_types=[
            pltpu.VMEM((IPW,), jnp.float32),
            pltpu.VMEM((IPW,), jnp.int32),
            pltpu.VMEM((IPW,), jnp.int32),
            pltpu.VMEM((IPW, 16), jnp.int32),
            pltpu.VMEM((IPW, 16), jnp.int32),
            pltpu.VMEM((IPW, 16), jnp.int32),
            pltpu.VMEM((IPW,), jnp.int32),
            pltpu.VMEM((IPW,), jnp.float32),
            pltpu.VMEM((IPW,), jnp.float32),
            pltpu.VMEM((IPW,), jnp.int32),
            pltpu.VMEM((IPW,), jnp.int32),
            pltpu.VMEM((IPW, D), jnp.float32),
            pltpu.VMEM((QPW, D), jnp.float32),
            pltpu.VMEM((QPW,), jnp.int32),
            pltpu.SemaphoreType.DMA,
        ],
    )
    def body(trs_h, tri_h, arg2d_h, ids2d_h, ents2d_h, keys_h, tid_h,
             ms_h, w_h, tv_h, ret_h, ent_h, gti_h, mask_h,
             trs_v, tri_v, rowidx_v, argrows_v, idrows_v, entrows_v,
             gti_v, ms_v, w_v, ent_v, mask_v, keys_v, ret_v, tid_v, sem):
        wid = lax.axis_index("s") * NC + lax.axis_index("c")
        ibase = wid * IPW
        qbase = wid * QPW

        pltpu.sync_copy(trs_h.at[pl.ds(ibase, IPW)], trs_v)
        pltpu.sync_copy(tri_h.at[pl.ds(ibase, IPW)], tri_v)
        pltpu.sync_copy(tid_h.at[pl.ds(qbase, QPW)], tid_v)

        lane = lax.iota(jnp.int32, 16)

        def l1(j, c):
            rid = tri_v[pl.ds(j * LANES, LANES)]
            qg = qbase + j // 2
            rowidx_v[pl.ds(j * LANES, LANES)] = (
                qg * (R // 16) + lax.shift_right_logical(rid, 2 + 2))
            return c
        lax.fori_loop(0, NVEC, l1, 0)
        pltpu.async_copy(arg2d_h.at[rowidx_v], argrows_v, sem).wait()

        def l2(j, c):
            rid = tri_v[pl.ds(j * LANES, LANES)]
            pos = j * LANES + lane
            a = plsc.load_gather(argrows_v, [pos, jnp.bitwise_and(rid, 15)])
            g = rid * V + a
            gti_v[pl.ds(j * LANES, LANES)] = g
            rowidx_v[pl.ds(j * LANES, LANES)] = lax.shift_right_logical(g, 4)
            return c
        lax.fori_loop(0, NVEC, l2, 0)
        pltpu.async_copy(ids2d_h.at[rowidx_v], idrows_v, sem).wait()
        pltpu.async_copy(ents2d_h.at[rowidx_v], entrows_v, sem).wait()
        pltpu.async_copy(keys_h.at[gti_v], keys_v, sem).wait()

        def l3(j, c):
            g = gti_v[pl.ds(j * LANES, LANES)]
            pos = j * LANES + lane
            lane16 = jnp.bitwise_and(g, 15)
            mid = plsc.load_gather(idrows_v, [pos, lane16])
            ent = plsc.load_gather(entrows_v, [pos, lane16])
            ent_v[pl.ds(j * LANES, LANES)] = ent
            tloc = jnp.zeros((16,), jnp.int32) + j // 2
            tb = plsc.load_gather(tid_v, [tloc])
            mk = (mid == tb).astype(jnp.int32)
            mask_v[pl.ds(j * LANES, LANES)] = mk
            s = trs_v[pl.ds(j * LANES, LANES)]
            ms_v[pl.ds(j * LANES, LANES)] = (
                s - mk.astype(jnp.float32) * LARGE_NUMBER)
            return c
        lax.fori_loop(0, NVEC, l3, 0)

        def l4(q, c):
            a = ms_v[pl.ds(q * K_TOP, 16)]
            b = ms_v[pl.ds(q * K_TOP + 16, 16)]
            m = jnp.max(jnp.maximum(a, b))
            ea = jnp.exp(a - m)
            eb = jnp.exp(b - m)
            s = jnp.sum(ea) + jnp.sum(eb)
            w_v[pl.ds(q * K_TOP, 16)] = ea / s
            w_v[pl.ds(q * K_TOP + 16, 16)] = eb / s
            return c
        lax.fori_loop(0, QPW, l4, 0)

        def l5(q, c):
            def inner(k2, acc):
                wk = plsc.load_gather(
                    w_v, [jnp.zeros((16,), jnp.int32) + (q * K_TOP + k2)])
                row = q * K_TOP + k2
                return tuple(
                    acc[ch] + keys_v[row, pl.ds(ch * 16, 16)] * wk
                    for ch in range(D // 16))
            acc0 = tuple(jnp.zeros((16,), jnp.float32)
                         for _ in range(D // 16))
            acc = lax.fori_loop(0, K_TOP, inner, acc0)
            for ch in range(D // 16):
                ret_v[q, pl.ds(ch * 16, 16)] = acc[ch]
            return c
        lax.fori_loop(0, QPW, l5, 0)

        pltpu.sync_copy(ms_v, ms_h.at[pl.ds(ibase, IPW)])
        pltpu.sync_copy(w_v, w_h.at[pl.ds(ibase, IPW)])
        pltpu.sync_copy(keys_v, tv_h.at[pl.ds(ibase, IPW)])
        pltpu.sync_copy(ret_v, ret_h.at[pl.ds(qbase, QPW)])
        pltpu.sync_copy(ent_v, ent_h.at[pl.ds(ibase, IPW)])
        pltpu.sync_copy(gti_v, gti_h.at[pl.ds(ibase, IPW)])
        pltpu.sync_copy(mask_v, mask_h.at[pl.ds(ibase, IPW)])

    return body(trs, tri, arg2d, ids2d, ents2d, flat_keys, text_ids)


def kernel(queries, memory_keys, memory_identifiers, memory_entity_ids,
           text_identifiers):
    flat_keys = memory_keys.reshape(R * V, D)
    ids2d = memory_identifiers.reshape(-1, 16)
    ents2d = memory_entity_ids.reshape(-1, 16)

    row_arg, trs, tri = _scores_topk(queries, memory_keys)
    arg2d = row_arg.reshape(-1, 16)

    ms, w, tv, ret, ent, gti, mk = _sc_retrieve(
        trs.reshape(-1), tri.reshape(-1), arg2d, ids2d, ents2d, flat_keys,
        text_identifiers)

    ms = ms.reshape(Q, K_TOP)
    w = w.reshape(Q, K_TOP)
    tv = tv.reshape(Q, K_TOP, D)
    ent = ent.reshape(Q, K_TOP)
    gti = gti.reshape(Q, K_TOP)
    mask = mk.reshape(Q, K_TOP).astype(jnp.bool_)
    return (ret, ms, w, tv, ent, gti, mask)


# speedup vs baseline: 4.6664x; 1.0147x over previous
import functools

import jax
import jax.numpy as jnp
from jax import lax
from jax.experimental import pallas as pl
from jax.experimental.pallas import tpu as pltpu
from jax.experimental.pallas import tpu_sc as plsc

Q = 256
R = 4096
V = 32
D = 128
K_TOP = 32
LARGE_NUMBER = 1e10
NEG = -1e30

RB = 256

NC = 2
NS = 16
LANES = 16
NW = NC * NS
QPW = Q // NW
IPW = QPW * K_TOP
NVEC = IPW // LANES


NB = R // RB


def _fused_body(*refs):
    q_ref = refs[0]
    key_refs = refs[1:1 + V]
    am_ref, s_ref, i_ref = refs[1 + V], refs[2 + V], refs[3 + V]
    vals3_ref = refs[4 + V]
    i_blk = pl.program_id(0)

    q = q_ref[...]
    m = None
    a = None
    for v in range(V):
        kv = key_refs[v][:, 0, 0, :]
        s = lax.dot_general(q, kv, (((1,), (1,)), ((), ())),
                            preferred_element_type=jnp.float32)
        if v == 0:
            m = s
            a = jnp.zeros(s.shape, jnp.int32)
        else:
            gt = s > m
            m = jnp.where(gt, s, m)
            a = jnp.where(gt, v, a)
    am_ref[...] = a
    vals3_ref[i_blk] = m

    @pl.when(i_blk == NB - 1)
    def _epilogue():
        vals = vals3_ref[...]
        colf3 = (lax.broadcasted_iota(jnp.int32, (NB, Q, RB), 0) * RB
                 + lax.broadcasted_iota(jnp.int32, (NB, Q, RB), 2)
                 ).astype(jnp.float32)
        bigf = jnp.float32(1e9)
        outs_s = []
        outs_i = []
        for _ in range(K_TOP):
            m1 = jnp.max(vals, axis=0)
            mm = jnp.max(m1, axis=1, keepdims=True)
            eq = vals == mm[None, :, :]
            t = jnp.where(eq, colf3, bigf)
            i1 = jnp.min(t, axis=0)
            idxf = jnp.min(i1, axis=1, keepdims=True)
            outs_s.append(mm)
            outs_i.append(idxf)
            vals = jnp.where(eq, NEG, vals)
        s_ref[...] = jnp.concatenate(outs_s, axis=1)
        i_ref[...] = jnp.concatenate(outs_i, axis=1).astype(jnp.int32)


def _keyspec(v):
    return pl.BlockSpec((RB, 1, 1, D), lambda i, _v=v: (i, _v, 0, 0))


def _scores_topk(queries, memory_keys):
    return pl.pallas_call(
        _fused_body,
        grid=(NB,),
        in_specs=[pl.BlockSpec((Q, D), lambda i: (0, 0))]
        + [_keyspec(v) for v in range(V)],
        out_specs=[
            pl.BlockSpec((Q, RB), lambda i: (0, i)),
            pl.BlockSpec((Q, K_TOP), lambda i: (0, 0)),
            pl.BlockSpec((Q, K_TOP), lambda i: (0, 0)),
        ],
        out_shape=[
            jax.ShapeDtypeStruct((Q, R), jnp.int32),
            jax.ShapeDtypeStruct((Q, K_TOP), jnp.float32),
            jax.ShapeDtypeStruct((Q, K_TOP), jnp.int32),
        ],
        scratch_shapes=[pltpu.VMEM((NB, Q, RB), jnp.float32)],
        compiler_params=pltpu.CompilerParams(
            dimension_semantics=("arbitrary",)),
    )(queries, *([memory_keys.reshape(R, V, 1, D)] * V))


def _sc_retrieve(trs, tri, arg2d, ids2d, ents2d, flat_keys, text_ids):
    mesh = plsc.VectorSubcoreMesh(core_axis_name="c", subcore_axis_name="s",
                                  num_cores=NC, num_subcores=NS)

    @functools.partial(
        pl.kernel,
        out_type=[
            jax.ShapeDtypeStruct((Q * K_TOP,), jnp.float32),
            jax.ShapeDtypeStruct((Q * K_TOP,), jnp.float32),
            jax.ShapeDtypeStruct((Q * K_TOP, D), jnp.float32),
            jax.ShapeDtypeStruct((Q, D), jnp.float32),
            jax.ShapeDtypeStruct((Q * K_TOP,), jnp.int32),
            jax.ShapeDtypeStruct((Q * K_TOP,), jnp.int32),
            jax.ShapeDtypeStruct((Q * K_TOP,), jnp.int32),
        ],
        mesh=mesh,
        compiler_params=pltpu.CompilerParams(needs_layout_passes=False,
                                             use_tc_tiling_on_sc=False),
        scratch_types=[
            pltpu.VMEM((IPW,), jnp.float32),
            pltpu.VMEM((IPW,), jnp.int32),
            pltpu.VMEM((IPW,), jnp.int32),
            pltpu.VMEM((IPW, 16), jnp.int32),
            pltpu.VMEM((IPW, 16), jnp.int32),
            pltpu.VMEM((IPW, 16), jnp.int32),
            pltpu.VMEM((IPW,), jnp.int32),
            pltpu.VMEM((IPW,), jnp.float32),
            pltpu.VMEM((IPW,), jnp.float32),
            pltpu.VMEM((IPW,), jnp.int32),
            pltpu.VMEM((IPW,), jnp.int32),
            pltpu.VMEM((IPW, D), jnp.float32),
            pltpu.VMEM((QPW, D), jnp.float32),
            pltpu.VMEM((QPW,), jnp.int32),
            pltpu.SemaphoreType.DMA,
            pltpu.SemaphoreType.DMA,
        ],
    )
    def body(trs_h, tri_h, arg2d_h, ids2d_h, ents2d_h, keys_h, tid_h,
             ms_h, w_h, tv_h, ret_h, ent_h, gti_h, mask_h,
             trs_v, tri_v, rowidx_v, argrows_v, idrows_v, entrows_v,
             gti_v, ms_v, w_v, ent_v, mask_v, keys_v, ret_v, tid_v, sem,
             sem2):
        wid = lax.axis_index("s") * NC + lax.axis_index("c")
        ibase = wid * IPW
        qbase = wid * QPW

        pltpu.sync_copy(trs_h.at[pl.ds(ibase, IPW)], trs_v)
        pltpu.sync_copy(tri_h.at[pl.ds(ibase, IPW)], tri_v)
        pltpu.sync_copy(tid_h.at[pl.ds(qbase, QPW)], tid_v)

        lane = lax.iota(jnp.int32, 16)

        def l1(j, c):
            rid = tri_v[pl.ds(j * LANES, LANES)]
            qg = qbase + j // 2
            rowidx_v[pl.ds(j * LANES, LANES)] = (
                qg * (R // 16) + lax.shift_right_logical(rid, 2 + 2))
            return c
        lax.fori_loop(0, NVEC, l1, 0)
        pltpu.async_copy(arg2d_h.at[rowidx_v], argrows_v, sem).wait()

        def l2(j, c):
            rid = tri_v[pl.ds(j * LANES, LANES)]
            pos = j * LANES + lane
            a = plsc.load_gather(argrows_v, [pos, jnp.bitwise_and(rid, 15)])
            g = rid * V + a
            gti_v[pl.ds(j * LANES, LANES)] = g
            rowidx_v[pl.ds(j * LANES, LANES)] = lax.shift_right_logical(g, 4)
            return c
        lax.fori_loop(0, NVEC, l2, 0)
        d_keys = pltpu.make_async_copy(keys_h.at[gti_v], keys_v, sem2)
        d_keys.start()
        d_ids = pltpu.make_async_copy(ids2d_h.at[rowidx_v], idrows_v, sem)
        d_ids.start()
        d_ents = pltpu.make_async_copy(ents2d_h.at[rowidx_v], entrows_v, sem)
        d_ents.start()
        d_ids.wait()
        d_ents.wait()

        def l3(j, c):
            g = gti_v[pl.ds(j * LANES, LANES)]
            pos = j * LANES + lane
            lane16 = jnp.bitwise_and(g, 15)
            mid = plsc.load_gather(idrows_v, [pos, lane16])
            ent = plsc.load_gather(entrows_v, [pos, lane16])
            ent_v[pl.ds(j * LANES, LANES)] = ent
            tloc = jnp.zeros((16,), jnp.int32) + j // 2
            tb = plsc.load_gather(tid_v, [tloc])
            mk = (mid == tb).astype(jnp.int32)
            mask_v[pl.ds(j * LANES, LANES)] = mk
            s = trs_v[pl.ds(j * LANES, LANES)]
            ms_v[pl.ds(j * LANES, LANES)] = (
                s - mk.astype(jnp.float32) * LARGE_NUMBER)
            return c
        lax.fori_loop(0, NVEC, l3, 0)

        def l4(q, c):
            a = ms_v[pl.ds(q * K_TOP, 16)]
            b = ms_v[pl.ds(q * K_TOP + 16, 16)]
            m = jnp.max(jnp.maximum(a, b))
            ea = jnp.exp(a - m)
            eb = jnp.exp(b - m)
            s = jnp.sum(ea) + jnp.sum(eb)
            w_v[pl.ds(q * K_TOP, 16)] = ea / s
            w_v[pl.ds(q * K_TOP + 16, 16)] = eb / s
            return c
        lax.fori_loop(0, QPW, l4, 0)

        d_keys.wait()

        def l5(q, c):
            def inner(k2, acc):
                wk = plsc.load_gather(
                    w_v, [jnp.zeros((16,), jnp.int32) + (q * K_TOP + k2)])
                row = q * K_TOP + k2
                return tuple(
                    acc[ch] + keys_v[row, pl.ds(ch * 16, 16)] * wk
                    for ch in range(D // 16))
            acc0 = tuple(jnp.zeros((16,), jnp.float32)
                         for _ in range(D // 16))
            acc = lax.fori_loop(0, K_TOP, inner, acc0)
            for ch in range(D // 16):
                ret_v[q, pl.ds(ch * 16, 16)] = acc[ch]
            return c
        lax.fori_loop(0, QPW, l5, 0)

        pltpu.sync_copy(ms_v, ms_h.at[pl.ds(ibase, IPW)])
        pltpu.sync_copy(w_v, w_h.at[pl.ds(ibase, IPW)])
        pltpu.sync_copy(keys_v, tv_h.at[pl.ds(ibase, IPW)])
        pltpu.sync_copy(ret_v, ret_h.at[pl.ds(qbase, QPW)])
        pltpu.sync_copy(ent_v, ent_h.at[pl.ds(ibase, IPW)])
        pltpu.sync_copy(gti_v, gti_h.at[pl.ds(ibase, IPW)])
        pltpu.sync_copy(mask_v, mask_h.at[pl.ds(ibase, IPW)])

    return body(trs, tri, arg2d, ids2d, ents2d, flat_keys, text_ids)


def kernel(queries, memory_keys, memory_identifiers, memory_entity_ids,
           text_identifiers):
    flat_keys = memory_keys.reshape(R * V, D)
    ids2d = memory_identifiers.reshape(-1, 16)
    ents2d = memory_entity_ids.reshape(-1, 16)

    row_arg, trs, tri = _scores_topk(queries, memory_keys)
    arg2d = row_arg.reshape(-1, 16)

    ms, w, tv, ret, ent, gti, mk = _sc_retrieve(
        trs.reshape(-1), tri.reshape(-1), arg2d, ids2d, ents2d, flat_keys,
        text_identifiers)

    ms = ms.reshape(Q, K_TOP)
    w = w.reshape(Q, K_TOP)
    tv = tv.reshape(Q, K_TOP, D)
    ent = ent.reshape(Q, K_TOP)
    gti = gti.reshape(Q, K_TOP)
    mask = mk.reshape(Q, K_TOP).astype(jnp.bool_)
    return (ret, ms, w, tv, ent, gti, mask)
